# baseline (speedup 1.0000x reference)
import jax
import jax.numpy as jnp
from jax import lax
from jax.experimental import pallas as pl
from jax.experimental.pallas import tpu as pltpu
from jax.experimental.pallas import tpu_sc as plsc

_DIM = 33
_DIM2 = _DIM * _DIM
_TBL = _DIM ** 3
_TBLP = 35944
_BINSIZE = 1.000001 / (_DIM - 1)
_W = 512
_CROWS = 2
_NCHUNK = 128 // _CROWS
_L = 16

_buf = lambda: pltpu.VMEM((_CROWS, _W), jnp.float32)


@pl.kernel(
    out_type=jax.ShapeDtypeStruct((8, 3, 512, 512), jnp.float32),
    mesh=plsc.VectorSubcoreMesh(core_axis_name="c", subcore_axis_name="s"),
    compiler_params=pltpu.CompilerParams(needs_layout_passes=False),
    scratch_types=[pltpu.VMEM((3 * _TBLP,), jnp.int32)]
    + [_buf() for _ in range(12)]
    + [pltpu.SemaphoreType.DMA for _ in range(4)],
)
def _lut3d_sc(lut_hbm, x_hbm, out_hbm, lut_v,
              i00, i01, i02, i10, i11, i12,
              o00, o01, o02, o10, o11, o12,
              sin0, sin1, sout0, sout1):
    wid = lax.axis_index("s") * 2 + lax.axis_index("c")
    img = wid // 4
    rows0 = (wid % 4) * 128

    ins = ((i00, i01, i02), (i10, i11, i12))
    outs = ((o00, o01, o02), (o10, o11, o12))
    sins = (sin0, sin1)
    souts = (sout0, sout1)

    def in_copies(ci, p):
        row = rows0 + ci * _CROWS
        return [pltpu.make_async_copy(
            x_hbm.at[img, c, pl.ds(row, _CROWS), :], ins[p][c], sins[p])
            for c in range(3)]

    def out_copies(ci, p):
        row = rows0 + ci * _CROWS
        return [pltpu.make_async_copy(
            outs[p][c], out_hbm.at[img, c, pl.ds(row, _CROWS), :], souts[p])
            for c in range(3)]

    pltpu.sync_copy(lut_hbm, lut_v)
    inv = 1.0 / _BINSIZE

    for cp in in_copies(0, 0):
        cp.start()

    @pl.loop(0, _NCHUNK, step=2)
    def chunk_pair(base):
        for b in range(2):
            ci = base + b
            p, q = b, 1 - b

            @pl.when(ci + 1 < _NCHUNK)
            def _prefetch():
                for cp in in_copies(ci + 1, q):
                    cp.start()

            for cp in in_copies(ci, p):
                cp.wait()

            @pl.when(ci >= 2)
            def _drain():
                for cp in out_copies(ci - 2, p):
                    cp.wait()

            rv, gv, bv = ins[p]
            ov0, ov1, ov2 = outs[p]

            @plsc.parallel_loop(0, _W, _L, unroll=4)
            def vec_body(s):
                for jj in range(_CROWS):
                    r = rv[jj, pl.ds(s, _L)]
                    g = gv[jj, pl.ds(s, _L)]
                    b_ = bv[jj, pl.ds(s, _L)]
                    rs = r * inv
                    gs = g * inv
                    bs = b_ * inv
                    ri = rs.astype(jnp.int32)
                    gi = gs.astype(jnp.int32)
                    bi = bs.astype(jnp.int32)
                    rd = rs - ri.astype(jnp.float32)
                    gd = gs - gi.astype(jnp.float32)
                    bd = bs - bi.astype(jnp.float32)
                    base_id = ri + gi * _DIM + bi * _DIM2

                    def gat(off, idx):
                        return plsc.load_gather(
                            lut_v.at[pl.ds(off, 3 * _TBLP - off)], [idx])

                    b1 = base_id + 1
                    b2 = base_id + 2
                    rm = 1.0 - rd
                    gm = 1.0 - gd
                    bm = 1.0 - bd
                    q00 = gm * bm
                    q10 = gd * bm
                    q01 = gm * bd
                    q11 = gd * bd
                    fmt = plsc.PackFormat.INTERLEAVED
                    wr = plsc.pack(rm, rd, format=fmt)
                    w00 = wr * plsc.pack(q00, q00, format=fmt)
                    w10 = wr * plsc.pack(q10, q10, format=fmt)
                    w01 = wr * plsc.pack(q01, q01, format=fmt)
                    w11 = wr * plsc.pack(q11, q11, format=fmt)
                    res = []
                    for c in range(3):
                        ct = c * _TBLP
                        p = (w00 * plsc.bitcast(gat(ct, base_id), jnp.bfloat16)
                             + w10 * plsc.bitcast(gat(ct + 32, b1),
                                                  jnp.bfloat16)) \
                            + (w01 * plsc.bitcast(gat(ct + 1088, b1),
                                                  jnp.bfloat16)
                               + w11 * plsc.bitcast(gat(ct + 1120, b2),
                                                    jnp.bfloat16))
                        ai = plsc.bitcast(p, jnp.int32)
                        lo = plsc.bitcast(ai << 16, jnp.float32)
                        hi = plsc.bitcast(ai & jnp.int32(-65536), jnp.float32)
                        res.append(lo + hi)
                    ov0[jj, pl.ds(s, _L)] = res[0]
                    ov1[jj, pl.ds(s, _L)] = res[1]
                    ov2[jj, pl.ds(s, _L)] = res[2]

            for cp in out_copies(ci, p):
                cp.start()

    for cp in out_copies(_NCHUNK - 2, 0):
        cp.wait()
    for cp in out_copies(_NCHUNK - 1, 1):
        cp.wait()


def _pack_pairs(lut):
    t = lut.reshape(3, _TBL)
    tn = jnp.concatenate([t[:, 1:], t[:, -1:]], axis=1)
    lo = lax.bitcast_convert_type(t.astype(jnp.bfloat16), jnp.uint16)
    hi = lax.bitcast_convert_type(tn.astype(jnp.bfloat16), jnp.uint16)
    packed = lo.astype(jnp.uint32) | (hi.astype(jnp.uint32) << 16)
    packed = jnp.pad(packed, ((0, 0), (0, _TBLP - _TBL)))
    return lax.bitcast_convert_type(packed, jnp.int32).reshape(3 * _TBLP)


def kernel(lut, x):
    return _lut3d_sc(_pack_pairs(lut), x)


# speedup vs baseline: 1.1213x; 1.1213x over previous
import jax
import jax.numpy as jnp
from jax import lax
from jax.experimental import pallas as pl
from jax.experimental.pallas import tpu as pltpu
from jax.experimental.pallas import tpu_sc as plsc

_DIM = 33
_DIM2 = _DIM * _DIM
_TBL = _DIM ** 3
_TBLP = 35944
_BINSIZE = 1.000001 / (_DIM - 1)
_W = 512
_CROWS = 2
_NCHUNK = 128 // _CROWS
_L = 16

_buf = lambda: pltpu.VMEM((_CROWS, _W), jnp.float32)


@pl.kernel(
    out_type=jax.ShapeDtypeStruct((8, 3, 512, 512), jnp.float32),
    mesh=plsc.VectorSubcoreMesh(core_axis_name="c", subcore_axis_name="s"),
    compiler_params=pltpu.CompilerParams(needs_layout_passes=False),
    scratch_types=[pltpu.VMEM((3 * _TBLP,), jnp.int32)]
    + [_buf() for _ in range(12)]
    + [pltpu.SemaphoreType.DMA for _ in range(4)],
)
def _lut3d_sc(lut_hbm, x_hbm, out_hbm, lut_v,
              i00, i01, i02, i10, i11, i12,
              o00, o01, o02, o10, o11, o12,
              sin0, sin1, sout0, sout1):
    wid = lax.axis_index("s") * 2 + lax.axis_index("c")
    img = wid // 4
    rows0 = (wid % 4) * 128

    ins = ((i00, i01, i02), (i10, i11, i12))
    outs = ((o00, o01, o02), (o10, o11, o12))
    sins = (sin0, sin1)
    souts = (sout0, sout1)

    def in_copies(ci, p):
        row = rows0 + ci * _CROWS
        return [pltpu.make_async_copy(
            x_hbm.at[img, c, pl.ds(row, _CROWS), :], ins[p][c], sins[p])
            for c in range(3)]

    def out_copies(ci, p):
        row = rows0 + ci * _CROWS
        return [pltpu.make_async_copy(
            outs[p][c], out_hbm.at[img, c, pl.ds(row, _CROWS), :], souts[p])
            for c in range(3)]

    pltpu.sync_copy(lut_hbm, lut_v)
    inv = 1.0 / _BINSIZE

    for cp in in_copies(0, 0):
        cp.start()

    @pl.loop(0, _NCHUNK, step=2)
    def chunk_pair(base):
        for b in range(2):
            ci = base + b
            p, q = b, 1 - b

            @pl.when(ci + 1 < _NCHUNK)
            def _prefetch():
                for cp in in_copies(ci + 1, q):
                    cp.start()

            for cp in in_copies(ci, p):
                cp.wait()

            @pl.when(ci >= 2)
            def _drain():
                for cp in out_copies(ci - 2, p):
                    cp.wait()

            rv, gv, bv = ins[p]
            ov0, ov1, ov2 = outs[p]

            @plsc.parallel_loop(0, _W, _L, unroll=2)
            def vec_body(s):
                for jj in range(_CROWS):
                    r = rv[jj, pl.ds(s, _L)]
                    g = gv[jj, pl.ds(s, _L)]
                    b_ = bv[jj, pl.ds(s, _L)]
                    rs = r * inv
                    gs = g * inv
                    bs = b_ * inv
                    ri = rs.astype(jnp.int32)
                    gi = gs.astype(jnp.int32)
                    bi = bs.astype(jnp.int32)
                    rd = rs - ri.astype(jnp.float32)
                    gd = gs - gi.astype(jnp.float32)
                    bd = bs - bi.astype(jnp.float32)
                    base_id = ri + gi * _DIM + bi * _DIM2

                    def gat(off, idx):
                        return plsc.load_gather(
                            lut_v.at[pl.ds(off, 3 * _TBLP - off)], [idx])

                    b1 = base_id + 1
                    b2 = base_id + 2
                    rm = 1.0 - rd
                    gm = 1.0 - gd
                    bm = 1.0 - bd
                    q00 = gm * bm
                    q10 = gd * bm
                    q01 = gm * bd
                    q11 = gd * bd
                    fmt = plsc.PackFormat.INTERLEAVED
                    wr = plsc.pack(rm, rd, format=fmt)
                    w00 = wr * plsc.pack(q00, q00, format=fmt)
                    w10 = wr * plsc.pack(q10, q10, format=fmt)
                    w01 = wr * plsc.pack(q01, q01, format=fmt)
                    w11 = wr * plsc.pack(q11, q11, format=fmt)
                    res = []
                    for c in range(3):
                        ct = c * _TBLP
                        p = (w00 * plsc.bitcast(gat(ct, base_id), jnp.bfloat16)
                             + w10 * plsc.bitcast(gat(ct + 32, b1),
                                                  jnp.bfloat16)) \
                            + (w01 * plsc.bitcast(gat(ct + 1088, b1),
                                                  jnp.bfloat16)
                               + w11 * plsc.bitcast(gat(ct + 1120, b2),
                                                    jnp.bfloat16))
                        ai = plsc.bitcast(p, jnp.int32)
                        lo = plsc.bitcast(ai << 16, jnp.float32)
                        hi = plsc.bitcast(ai, jnp.float32)
                        res.append(lo + hi)
                    ov0[jj, pl.ds(s, _L)] = res[0]
                    ov1[jj, pl.ds(s, _L)] = res[1]
                    ov2[jj, pl.ds(s, _L)] = res[2]

            for cp in out_copies(ci, p):
                cp.start()

    for cp in out_copies(_NCHUNK - 2, 0):
        cp.wait()
    for cp in out_copies(_NCHUNK - 1, 1):
        cp.wait()


def _pack_pairs(lut):
    t = lut.reshape(3, _TBL)
    tn = jnp.concatenate([t[:, 1:], t[:, -1:]], axis=1)
    lo = lax.bitcast_convert_type(t.astype(jnp.bfloat16), jnp.uint16)
    hi = lax.bitcast_convert_type(tn.astype(jnp.bfloat16), jnp.uint16)
    packed = lo.astype(jnp.uint32) | (hi.astype(jnp.uint32) << 16)
    packed = jnp.pad(packed, ((0, 0), (0, _TBLP - _TBL)))
    return lax.bitcast_convert_type(packed, jnp.int32).reshape(3 * _TBLP)


def kernel(lut, x):
    return _lut3d_sc(_pack_pairs(lut), x)


# speedup vs baseline: 1.1309x; 1.0085x over previous
import jax
import jax.numpy as jnp
from jax import lax
from jax.experimental import pallas as pl
from jax.experimental.pallas import tpu as pltpu
from jax.experimental.pallas import tpu_sc as plsc

_DIM = 33
_DIM2 = _DIM * _DIM
_TBL = _DIM ** 3
_TBLP = 35944
_BINSIZE = 1.000001 / (_DIM - 1)
_W = 512
_CROWS = 2
_NCHUNK = 128 // _CROWS
_L = 16

_buf = lambda: pltpu.VMEM((_CROWS, _W), jnp.float32)


@pl.kernel(
    out_type=jax.ShapeDtypeStruct((8, 3, 512, 512), jnp.float32),
    mesh=plsc.VectorSubcoreMesh(core_axis_name="c", subcore_axis_name="s"),
    compiler_params=pltpu.CompilerParams(needs_layout_passes=False),
    scratch_types=[pltpu.VMEM((3 * _TBLP,), jnp.int32)]
    + [_buf() for _ in range(12)]
    + [pltpu.SemaphoreType.DMA for _ in range(4)],
)
def _lut3d_sc(lut_hbm, x_hbm, out_hbm, lut_v,
              i00, i01, i02, i10, i11, i12,
              o00, o01, o02, o10, o11, o12,
              sin0, sin1, sout0, sout1):
    wid = lax.axis_index("s") * 2 + lax.axis_index("c")
    img = wid // 4
    rows0 = (wid % 4) * 128

    ins = ((i00, i01, i02), (i10, i11, i12))
    outs = ((o00, o01, o02), (o10, o11, o12))
    sins = (sin0, sin1)
    souts = (sout0, sout1)

    def in_copies(ci, p):
        row = rows0 + ci * _CROWS
        return [pltpu.make_async_copy(
            x_hbm.at[img, c, pl.ds(row, _CROWS), :], ins[p][c], sins[p])
            for c in range(3)]

    def out_copies(ci, p):
        row = rows0 + ci * _CROWS
        return [pltpu.make_async_copy(
            outs[p][c], out_hbm.at[img, c, pl.ds(row, _CROWS), :], souts[p])
            for c in range(3)]

    for cp in in_copies(0, 0):
        cp.start()
    pltpu.sync_copy(lut_hbm, lut_v)
    inv = 1.0 / _BINSIZE

    @pl.loop(0, _NCHUNK, step=2)
    def chunk_pair(base):
        for b in range(2):
            ci = base + b
            p, q = b, 1 - b

            @pl.when(ci + 1 < _NCHUNK)
            def _prefetch():
                for cp in in_copies(ci + 1, q):
                    cp.start()

            for cp in in_copies(ci, p):
                cp.wait()

            @pl.when(ci >= 2)
            def _drain():
                for cp in out_copies(ci - 2, p):
                    cp.wait()

            rv, gv, bv = ins[p]
            ov0, ov1, ov2 = outs[p]

            @plsc.parallel_loop(0, _W, _L, unroll=2)
            def vec_body(s):
                for jj in range(_CROWS):
                    r = rv[jj, pl.ds(s, _L)]
                    g = gv[jj, pl.ds(s, _L)]
                    b_ = bv[jj, pl.ds(s, _L)]
                    rs = r * inv
                    gs = g * inv
                    bs = b_ * inv
                    ri = rs.astype(jnp.int32)
                    gi = gs.astype(jnp.int32)
                    bi = bs.astype(jnp.int32)
                    rd = rs - ri.astype(jnp.float32)
                    gd = gs - gi.astype(jnp.float32)
                    bd = bs - bi.astype(jnp.float32)
                    base_id = ri + gi * _DIM + bi * _DIM2

                    def gat(off, idx):
                        return plsc.load_gather(
                            lut_v.at[pl.ds(off, 3 * _TBLP - off)], [idx])

                    b1 = base_id + 1
                    b2 = base_id + 2
                    rm = 1.0 - rd
                    gm = 1.0 - gd
                    bm = 1.0 - bd
                    q00 = gm * bm
                    q10 = gd * bm
                    q01 = gm * bd
                    q11 = gd * bd
                    fmt = plsc.PackFormat.INTERLEAVED
                    wr = plsc.pack(rm, rd, format=fmt)
                    w00 = wr * plsc.pack(q00, q00, format=fmt)
                    w10 = wr * plsc.pack(q10, q10, format=fmt)
                    w01 = wr * plsc.pack(q01, q01, format=fmt)
                    w11 = wr * plsc.pack(q11, q11, format=fmt)
                    res = []
                    for c in range(3):
                        ct = c * _TBLP
                        p = (w00 * plsc.bitcast(gat(ct, base_id), jnp.bfloat16)
                             + w10 * plsc.bitcast(gat(ct + 32, b1),
                                                  jnp.bfloat16)) \
                            + (w01 * plsc.bitcast(gat(ct + 1088, b1),
                                                  jnp.bfloat16)
                               + w11 * plsc.bitcast(gat(ct + 1120, b2),
                                                    jnp.bfloat16))
                        ai = plsc.bitcast(p, jnp.int32)
                        lo = plsc.bitcast(ai << 16, jnp.float32)
                        hi = plsc.bitcast(ai, jnp.float32)
                        res.append(lo + hi)
                    ov0[jj, pl.ds(s, _L)] = res[0]
                    ov1[jj, pl.ds(s, _L)] = res[1]
                    ov2[jj, pl.ds(s, _L)] = res[2]

            for cp in out_copies(ci, p):
                cp.start()

    for cp in out_copies(_NCHUNK - 2, 0):
        cp.wait()
    for cp in out_copies(_NCHUNK - 1, 1):
        cp.wait()


def _pack_pairs(lut):
    t = lut.reshape(3, _TBL)
    tn = jnp.concatenate([t[:, 1:], t[:, -1:]], axis=1)
    lo = lax.bitcast_convert_type(t.astype(jnp.bfloat16), jnp.uint16)
    hi = lax.bitcast_convert_type(tn.astype(jnp.bfloat16), jnp.uint16)
    packed = lo.astype(jnp.uint32) | (hi.astype(jnp.uint32) << 16)
    packed = jnp.pad(packed, ((0, 0), (0, _TBLP - _TBL)))
    return lax.bitcast_convert_type(packed, jnp.int32).reshape(3 * _TBLP)


def kernel(lut, x):
    return _lut3d_sc(_pack_pairs(lut), x)
